# baseline (speedup 1.0000x reference)
import functools

import jax
import jax.numpy as jnp
from jax import lax
from jax.experimental import pallas as pl
from jax.experimental.pallas import tpu as pltpu
from jax.experimental.pallas import tpu_sc as plsc

_D = 128
_NN = 10000
_NE = 320000
_NW = 32
_EPW = 10240
_NEP = _EPW * _NW
_K = 40
_NCH = _EPW // _K
_NNP = 10240
_RPT = _NNP // 16


@functools.lru_cache(maxsize=None)
def _make_edge_kernel():
  mesh = plsc.VectorSubcoreMesh(core_axis_name="c", subcore_axis_name="s",
                                num_cores=2, num_subcores=16)

  @functools.partial(
      pl.kernel,
      mesh=mesh,
      out_type=jax.ShapeDtypeStruct((2, _NNP, _D), jnp.float32),
      scratch_types=[
          [pltpu.VMEM((_K,), jnp.int32)] * 4,
          [pltpu.VMEM((_K,), jnp.int32)] * 4,
          [pltpu.VMEM((_K, _D), jnp.float32)] * 4,
          [pltpu.VMEM((_K, _D), jnp.float32)] * 4,
          pltpu.VMEM_SHARED((_NNP, _D), jnp.float32),
          [pltpu.SemaphoreType.DMA] * 4,
          [pltpu.SemaphoreType.DMA] * 4,
          [pltpu.SemaphoreType.DMA] * 4,
      ],
  )
  def edge_kernel(pa_hbm, pb_hbm, src_hbm, dst_hbm, zeros_hbm,
                  acc_hbm, si, di, ra, rb, acc_sh, isem, gsem, ssem):
    cid = lax.axis_index("c")
    sid = lax.axis_index("s")
    wid = sid * 2 + cid

    pltpu.sync_copy(zeros_hbm, acc_sh.at[pl.ds(sid * _RPT, _RPT)])
    plsc.subcore_barrier()

    def load_idx(c, b):
      base = wid * _EPW + c * _K
      pltpu.async_copy(src_hbm.at[pl.ds(base, _K)], si[b], isem[b])
      pltpu.async_copy(dst_hbm.at[pl.ds(base, _K)], di[b], isem[b])

    def wait_idx(c, b):
      base = wid * _EPW + c * _K
      pltpu.make_async_copy(src_hbm.at[pl.ds(base, _K)], si[b],
                            isem[b]).wait()
      pltpu.make_async_copy(dst_hbm.at[pl.ds(base, _K)], di[b],
                            isem[b]).wait()

    def issue_gathers(b):
      pltpu.async_copy(pa_hbm.at[si[b]], ra[b], gsem[b])
      pltpu.async_copy(pb_hbm.at[di[b]], rb[b], gsem[b])

    def wait_gathers(b):
      pltpu.make_async_copy(pa_hbm.at[si[b]], ra[b], gsem[b]).wait()
      pltpu.make_async_copy(pb_hbm.at[di[b]], rb[b], gsem[b]).wait()

    def compute(b):
      rab, rbb = ra[b], rb[b]

      def group(g, gcarry):
        for k in range(8):
          e = g * 8 + k
          for r in range(_D // 16):
            sl = pl.ds(r * 16, 16)
            s = rab[e, sl] + rbb[e, sl]
            rab[e, sl] = s / (jnp.float32(-1.0) - jnp.exp(s))
        return gcarry

      lax.fori_loop(0, _K // 8, group, 0)

    def issue_scatters(b):
      pltpu.async_copy(ra[b], acc_sh.at[di[b]], ssem[b], add=True)

    def wait_scatters(b):
      pltpu.make_async_copy(ra[b], acc_sh.at[di[b]], ssem[b]).wait()

    load_idx(0, 0)
    load_idx(1, 1)
    load_idx(2, 2)
    wait_idx(0, 0)
    issue_gathers(0)
    wait_idx(1, 1)
    issue_gathers(1)

    def step(c, b, load_c, gather_c, drain_b):
      wait_gathers(b)
      compute(b)
      issue_scatters(b)
      if drain_b is not None:
        wait_scatters(drain_b)
      if load_c is not None:
        load_idx(load_c[0], load_c[1])
      if gather_c is not None:
        wait_idx(gather_c[0], gather_c[1])
        issue_gathers(gather_c[1])

    step(0, 0, (3, 3), (2, 2), None)
    step(1, 1, (4, 0), (3, 3), 0)
    step(2, 2, (5, 1), (4, 0), 1)
    step(3, 3, (6, 2), (5, 1), 2)

    def macro(j, carry):
      c = 4 * j
      step(c + 0, 0, (c + 3, 3), (c + 2, 2), 3)
      step(c + 1, 1, (c + 4, 0), (c + 3, 3), 0)
      step(c + 2, 2, (c + 5, 1), (c + 4, 0), 1)
      step(c + 3, 3, (c + 6, 2), (c + 5, 1), 2)
      return carry

    lax.fori_loop(1, _NCH // 4 - 1, macro, 0)

    c = _NCH - 4
    step(c + 0, 0, (c + 3, 3), (c + 2, 2), 3)
    step(c + 1, 1, None, (c + 3, 3), 0)
    step(c + 2, 2, None, None, 1)
    step(c + 3, 3, None, None, 2)
    wait_scatters(3)
    plsc.subcore_barrier()

    sl = pl.ds(sid * _RPT, _RPT)
    pltpu.sync_copy(acc_sh.at[sl], acc_hbm.at[cid, sl])

  return edge_kernel


_ROWS = 1000
_EB = 4096


def _pre_body(h_ref, wcat_ref, b1_ref, pa_ref, pb_ref):
  p = jnp.dot(h_ref[...], wcat_ref[...], preferred_element_type=jnp.float32)
  pa_ref[...] = p[:, :_D]
  pb_ref[...] = p[:, _D:] + b1_ref[...]


def _hist_body(d_ref, hist_ref):
  i = pl.program_id(0)
  d = d_ref[...]
  lanes = lax.broadcasted_iota(jnp.int32, (_EB, _D), 1)
  ohhi = (lax.shift_right_logical(d, 7) == lanes).astype(jnp.float32)
  ohlo = (lax.bitwise_and(d, 127) == lanes).astype(jnp.float32)
  p = lax.dot_general(ohhi, ohlo, (((0,), (0,)), ((), ())),
                      preferred_element_type=jnp.float32)

  @pl.when(i == 0)
  def _():
    hist_ref[...] = p

  @pl.when(i > 0)
  def _():
    hist_ref[...] += p


def _post_body(acc_ref, cnt_ref, h_ref, w2_ref, b2_ref, u1_ref, ub1_ref,
               u2_ref, ub2_ref, out_ref):
  accs = acc_ref[0] + acc_ref[1]
  agg = (jnp.dot(accs, w2_ref[...], preferred_element_type=jnp.float32)
         + cnt_ref[...] * b2_ref[...])
  u1 = u1_ref[...]
  u = (jnp.dot(h_ref[...], u1[:_D], preferred_element_type=jnp.float32)
       + jnp.dot(agg, u1[_D:], preferred_element_type=jnp.float32)
       + ub1_ref[...])
  t = u / (1.0 + jnp.exp(-u))
  out_ref[...] = (jnp.dot(t, u2_ref[...], preferred_element_type=jnp.float32)
                  + ub2_ref[...])


def kernel(h, edge_index, msg_w1, msg_b1, msg_w2, msg_b2,
           upd_w1, upd_b1, upd_w2, upd_b2):
  src = edge_index[0].astype(jnp.int32)
  dst = edge_index[1].astype(jnp.int32)
  pad = _NEP - _NE
  src_p = jnp.concatenate([src, jnp.zeros((pad,), jnp.int32)])
  dst_p = jnp.concatenate([dst, jnp.full((pad,), _NNP - 1, jnp.int32)])

  wcat = -jnp.concatenate([msg_w1[:_D], msg_w1[_D:]], axis=1)
  msg_b1 = -msg_b1

  pa, pb = pl.pallas_call(
      _pre_body,
      grid=(_NN // _ROWS,),
      in_specs=[
          pl.BlockSpec((_ROWS, _D), lambda i: (i, 0)),
          pl.BlockSpec((_D, 2 * _D), lambda i: (0, 0)),
          pl.BlockSpec((1, _D), lambda i: (0, 0)),
      ],
      out_specs=[
          pl.BlockSpec((_ROWS, _D), lambda i: (i, 0)),
          pl.BlockSpec((_ROWS, _D), lambda i: (i, 0)),
      ],
      out_shape=[
          jax.ShapeDtypeStruct((_NN, _D), jnp.float32),
          jax.ShapeDtypeStruct((_NN, _D), jnp.float32),
      ],
  )(h, wcat, msg_b1[None, :])

  zeros = jnp.zeros((_RPT, _D), jnp.float32)
  acc = _make_edge_kernel()(pa, pb, src_p, dst_p, zeros)

  hist = pl.pallas_call(
      _hist_body,
      grid=(_NEP // _EB,),
      in_specs=[pl.BlockSpec((_EB, 1), lambda i: (i, 0))],
      out_specs=pl.BlockSpec((_D, _D), lambda i: (0, 0)),
      out_shape=jax.ShapeDtypeStruct((_D, _D), jnp.float32),
  )(dst_p.reshape(_NEP, 1))
  cnt_col = hist.reshape(_D * _D)[:_NN][:, None]

  out = pl.pallas_call(
      _post_body,
      grid=(_NN // _ROWS,),
      in_specs=[
          pl.BlockSpec((2, _ROWS, _D), lambda i: (0, i, 0)),
          pl.BlockSpec((_ROWS, 1), lambda i: (i, 0)),
          pl.BlockSpec((_ROWS, _D), lambda i: (i, 0)),
          pl.BlockSpec((_D, _D), lambda i: (0, 0)),
          pl.BlockSpec((1, _D), lambda i: (0, 0)),
          pl.BlockSpec((2 * _D, _D), lambda i: (0, 0)),
          pl.BlockSpec((1, _D), lambda i: (0, 0)),
          pl.BlockSpec((_D, _D), lambda i: (0, 0)),
          pl.BlockSpec((1, _D), lambda i: (0, 0)),
      ],
      out_specs=pl.BlockSpec((_ROWS, _D), lambda i: (i, 0)),
      out_shape=jax.ShapeDtypeStruct((_NN, _D), jnp.float32),
  )(acc, cnt_col, h, msg_w2, msg_b2[None, :], upd_w1, upd_b1[None, :],
    upd_w2, upd_b2[None, :])
  return out


# speedup vs baseline: 1.0204x; 1.0204x over previous
import functools

import jax
import jax.numpy as jnp
from jax import lax
from jax.experimental import pallas as pl
from jax.experimental.pallas import tpu as pltpu
from jax.experimental.pallas import tpu_sc as plsc

_D = 128
_NN = 10000
_NE = 320000
_NW = 32
_EPW = 10240
_NEP = _EPW * _NW
_K = 40
_NCH = _EPW // _K
_NNP = 10240
_RPT = _NNP // 16


@functools.lru_cache(maxsize=None)
def _make_edge_kernel():
  mesh = plsc.VectorSubcoreMesh(core_axis_name="c", subcore_axis_name="s",
                                num_cores=2, num_subcores=16)

  @functools.partial(
      pl.kernel,
      mesh=mesh,
      out_type=jax.ShapeDtypeStruct((2, _NNP, _D), jnp.float32),
      scratch_types=[
          [pltpu.VMEM((_K,), jnp.int32)] * 4,
          [pltpu.VMEM((_K,), jnp.int32)] * 4,
          [pltpu.VMEM((_K, _D), jnp.float32)] * 4,
          [pltpu.VMEM((_K, _D), jnp.float32)] * 4,
          pltpu.VMEM_SHARED((_NNP, _D), jnp.float32),
          [pltpu.SemaphoreType.DMA] * 4,
          [pltpu.SemaphoreType.DMA] * 4,
          [pltpu.SemaphoreType.DMA] * 4,
      ],
  )
  def edge_kernel(pa_hbm, pb_hbm, src_hbm, dst_hbm, zeros_hbm,
                  acc_hbm, si, di, ra, rb, acc_sh, isem, gsem, ssem):
    cid = lax.axis_index("c")
    sid = lax.axis_index("s")
    wid = sid * 2 + cid

    pltpu.sync_copy(zeros_hbm, acc_sh.at[pl.ds(sid * _RPT, _RPT)])
    plsc.subcore_barrier()

    def load_idx(c, b):
      base = wid * _EPW + c * _K
      pltpu.async_copy(src_hbm.at[pl.ds(base, _K)], si[b], isem[b])
      pltpu.async_copy(dst_hbm.at[pl.ds(base, _K)], di[b], isem[b])

    def wait_idx(c, b):
      base = wid * _EPW + c * _K
      pltpu.make_async_copy(src_hbm.at[pl.ds(base, _K)], si[b],
                            isem[b]).wait()
      pltpu.make_async_copy(dst_hbm.at[pl.ds(base, _K)], di[b],
                            isem[b]).wait()

    def issue_gathers(b):
      pltpu.async_copy(pa_hbm.at[si[b]], ra[b], gsem[b])
      pltpu.async_copy(pb_hbm.at[di[b]], rb[b], gsem[b])

    def wait_gathers(b):
      pltpu.make_async_copy(pa_hbm.at[si[b]], ra[b], gsem[b]).wait()
      pltpu.make_async_copy(pb_hbm.at[di[b]], rb[b], gsem[b]).wait()

    def compute(b):
      rab, rbb = ra[b], rb[b]

      def group(g, gcarry):
        for k in range(8):
          e = g * 8 + k
          for r in range(_D // 16):
            sl = pl.ds(r * 16, 16)
            s = rab[e, sl] + rbb[e, sl]
            rab[e, sl] = s / (jnp.float32(-1.0) - jnp.exp(s))
        return gcarry

      lax.fori_loop(0, _K // 8, group, 0)

    def issue_scatters(b):
      pltpu.async_copy(ra[b], acc_sh.at[di[b]], ssem[b], add=True)

    def wait_scatters(b):
      pltpu.make_async_copy(ra[b], acc_sh.at[di[b]], ssem[b]).wait()

    load_idx(0, 0)
    load_idx(1, 1)
    load_idx(2, 2)
    wait_idx(0, 0)
    issue_gathers(0)
    wait_idx(1, 1)
    issue_gathers(1)

    def step(c, b, load_c, gather_c, drain_b):
      wait_gathers(b)
      compute(b)
      issue_scatters(b)
      if drain_b is not None:
        wait_scatters(drain_b)
      if load_c is not None:
        load_idx(load_c[0], load_c[1])
      if gather_c is not None:
        wait_idx(gather_c[0], gather_c[1])
        issue_gathers(gather_c[1])

    step(0, 0, (3, 3), (2, 2), None)
    step(1, 1, (4, 0), (3, 3), 0)
    step(2, 2, (5, 1), (4, 0), 1)
    step(3, 3, (6, 2), (5, 1), 2)

    def macro(j, carry):
      c = 4 * j
      step(c + 0, 0, (c + 3, 3), (c + 2, 2), 3)
      step(c + 1, 1, (c + 4, 0), (c + 3, 3), 0)
      step(c + 2, 2, (c + 5, 1), (c + 4, 0), 1)
      step(c + 3, 3, (c + 6, 2), (c + 5, 1), 2)
      return carry

    lax.fori_loop(1, _NCH // 4 - 1, macro, 0)

    c = _NCH - 4
    step(c + 0, 0, (c + 3, 3), (c + 2, 2), 3)
    step(c + 1, 1, None, (c + 3, 3), 0)
    step(c + 2, 2, None, None, 1)
    step(c + 3, 3, None, None, 2)
    wait_scatters(3)
    plsc.subcore_barrier()

    sl = pl.ds(sid * _RPT, _RPT)
    pltpu.sync_copy(acc_sh.at[sl], acc_hbm.at[cid, sl])

  return edge_kernel


_ROWS = 1000
_EB = 4096


def _pre_body(h_ref, wcat_ref, b1_ref, pa_ref, pb_ref):
  p = jnp.dot(h_ref[...], wcat_ref[...], preferred_element_type=jnp.float32)
  pa_ref[...] = p[:, :_D]
  pb_ref[...] = p[:, _D:] + b1_ref[...]


def _hist_body(d_ref, hist_ref):
  i = pl.program_id(0)
  d = d_ref[...]
  lanes = lax.broadcasted_iota(jnp.int32, (_EB, _D), 1)
  ohhi = (lax.shift_right_logical(d, 7) == lanes).astype(jnp.float32)
  ohlo = (lax.bitwise_and(d, 127) == lanes).astype(jnp.float32)
  p = lax.dot_general(ohhi, ohlo, (((0,), (0,)), ((), ())),
                      preferred_element_type=jnp.float32)

  @pl.when(i == 0)
  def _():
    hist_ref[...] = p

  @pl.when(i > 0)
  def _():
    hist_ref[...] += p


def _post_body(acc_ref, cnt_ref, h_ref, w2_ref, b2_ref, u1_ref, ub1_ref,
               u2_ref, ub2_ref, out_ref):
  accs = acc_ref[0] + acc_ref[1]
  agg = (jnp.dot(accs, w2_ref[...], preferred_element_type=jnp.float32)
         + cnt_ref[...] * b2_ref[...])
  u1 = u1_ref[...]
  u = (jnp.dot(h_ref[...], u1[:_D], preferred_element_type=jnp.float32)
       + jnp.dot(agg, u1[_D:], preferred_element_type=jnp.float32)
       + ub1_ref[...])
  t = u / (1.0 + jnp.exp(-u))
  out_ref[...] = (jnp.dot(t, u2_ref[...], preferred_element_type=jnp.float32)
                  + ub2_ref[...])


def kernel(h, edge_index, msg_w1, msg_b1, msg_w2, msg_b2,
           upd_w1, upd_b1, upd_w2, upd_b2):
  src = edge_index[0].astype(jnp.int32)
  dst = edge_index[1].astype(jnp.int32)
  pad = _NEP - _NE
  src_p = jnp.concatenate([src, jnp.zeros((pad,), jnp.int32)])
  junk = _NN + jnp.arange(pad, dtype=jnp.int32) % (_NNP - _NN)
  dst_p = jnp.concatenate([dst, junk])

  wcat = -jnp.concatenate([msg_w1[:_D], msg_w1[_D:]], axis=1)
  msg_b1 = -msg_b1

  pa, pb = pl.pallas_call(
      _pre_body,
      grid=(_NN // _ROWS,),
      in_specs=[
          pl.BlockSpec((_ROWS, _D), lambda i: (i, 0)),
          pl.BlockSpec((_D, 2 * _D), lambda i: (0, 0)),
          pl.BlockSpec((1, _D), lambda i: (0, 0)),
      ],
      out_specs=[
          pl.BlockSpec((_ROWS, _D), lambda i: (i, 0)),
          pl.BlockSpec((_ROWS, _D), lambda i: (i, 0)),
      ],
      out_shape=[
          jax.ShapeDtypeStruct((_NN, _D), jnp.float32),
          jax.ShapeDtypeStruct((_NN, _D), jnp.float32),
      ],
  )(h, wcat, msg_b1[None, :])

  zeros = jnp.zeros((_RPT, _D), jnp.float32)
  acc = _make_edge_kernel()(pa, pb, src_p, dst_p, zeros)

  hist = pl.pallas_call(
      _hist_body,
      grid=(_NEP // _EB,),
      in_specs=[pl.BlockSpec((_EB, 1), lambda i: (i, 0))],
      out_specs=pl.BlockSpec((_D, _D), lambda i: (0, 0)),
      out_shape=jax.ShapeDtypeStruct((_D, _D), jnp.float32),
  )(dst_p.reshape(_NEP, 1))
  cnt_col = hist.reshape(_D * _D)[:_NN][:, None]

  out = pl.pallas_call(
      _post_body,
      grid=(_NN // _ROWS,),
      in_specs=[
          pl.BlockSpec((2, _ROWS, _D), lambda i: (0, i, 0)),
          pl.BlockSpec((_ROWS, 1), lambda i: (i, 0)),
          pl.BlockSpec((_ROWS, _D), lambda i: (i, 0)),
          pl.BlockSpec((_D, _D), lambda i: (0, 0)),
          pl.BlockSpec((1, _D), lambda i: (0, 0)),
          pl.BlockSpec((2 * _D, _D), lambda i: (0, 0)),
          pl.BlockSpec((1, _D), lambda i: (0, 0)),
          pl.BlockSpec((_D, _D), lambda i: (0, 0)),
          pl.BlockSpec((1, _D), lambda i: (0, 0)),
      ],
      out_specs=pl.BlockSpec((_ROWS, _D), lambda i: (i, 0)),
      out_shape=jax.ShapeDtypeStruct((_NN, _D), jnp.float32),
  )(acc, cnt_col, h, msg_w2, msg_b2[None, :], upd_w1, upd_b1[None, :],
    upd_w2, upd_b2[None, :])
  return out


# speedup vs baseline: 1.9698x; 1.9305x over previous
import functools

import jax
import jax.numpy as jnp
from jax import lax
from jax.experimental import pallas as pl
from jax.experimental.pallas import tpu as pltpu
from jax.experimental.pallas import tpu_sc as plsc

_D = 128
_NN = 10000
_NE = 320000
_NW = 32
_EPW = 10240
_NEP = _EPW * _NW
_K = 40
_NCH = _EPW // _K
_NNP = 10240
_RPT = _NNP // 16


@functools.lru_cache(maxsize=None)
def _make_edge_kernel():
  mesh = plsc.VectorSubcoreMesh(core_axis_name="c", subcore_axis_name="s",
                                num_cores=2, num_subcores=16)

  @functools.partial(
      pl.kernel,
      mesh=mesh,
      out_type=jax.ShapeDtypeStruct((2, _NNP, _D), jnp.float32),
      scratch_types=[
          [pltpu.VMEM((_K,), jnp.int32)] * 4,
          [pltpu.VMEM((_K,), jnp.int32)] * 4,
          [pltpu.VMEM((_K, _D), jnp.float32)] * 4,
          [pltpu.VMEM((_K, _D), jnp.float32)] * 4,
          pltpu.VMEM_SHARED((_NNP, _D), jnp.float32),
          [pltpu.SemaphoreType.DMA] * 4,
          [pltpu.SemaphoreType.DMA] * 4,
          [pltpu.SemaphoreType.DMA] * 4,
      ],
  )
  def edge_kernel(pa_hbm, pb_hbm, src_hbm, dst_hbm, zeros_hbm,
                  acc_hbm, si, di, ra, rb, acc_sh, isem, gsem, ssem):
    cid = lax.axis_index("c")
    sid = lax.axis_index("s")
    wid = sid * 2 + cid

    pltpu.sync_copy(zeros_hbm, acc_sh.at[pl.ds(sid * _RPT, _RPT)])
    plsc.subcore_barrier()

    def load_idx(c, b):
      base = wid * _EPW + c * _K
      pltpu.async_copy(src_hbm.at[pl.ds(base, _K)], si[b], isem[b])
      pltpu.async_copy(dst_hbm.at[pl.ds(base, _K)], di[b], isem[b])

    def wait_idx(c, b):
      base = wid * _EPW + c * _K
      pltpu.make_async_copy(src_hbm.at[pl.ds(base, _K)], si[b],
                            isem[b]).wait()
      pltpu.make_async_copy(dst_hbm.at[pl.ds(base, _K)], di[b],
                            isem[b]).wait()

    def issue_gathers(b):
      pltpu.async_copy(pa_hbm.at[si[b]], ra[b], gsem[b])
      pltpu.async_copy(pb_hbm.at[di[b]], rb[b], gsem[b])

    def wait_gathers(b):
      pltpu.make_async_copy(pa_hbm.at[si[b]], ra[b], gsem[b]).wait()
      pltpu.make_async_copy(pb_hbm.at[di[b]], rb[b], gsem[b]).wait()

    def compute(b):
      rab, rbb = ra[b], rb[b]

      def group(g, gcarry):
        for k in range(8):
          e = g * 8 + k
          for r in range(_D // 16):
            sl = pl.ds(r * 16, 16)
            s = rab[e, sl] + rbb[e, sl]
            rab[e, sl] = s / (jnp.float32(-1.0) - jnp.exp(s))
        return gcarry

      lax.fori_loop(0, _K // 8, group, 0)

    def issue_scatters(b):
      pltpu.async_copy(ra[b], acc_sh.at[di[b]], ssem[b], add=True)

    def wait_scatters(b):
      pltpu.make_async_copy(ra[b], acc_sh.at[di[b]], ssem[b]).wait()

    load_idx(0, 0)
    load_idx(1, 1)
    load_idx(2, 2)
    wait_idx(0, 0)
    issue_gathers(0)
    wait_idx(1, 1)
    issue_gathers(1)

    def step(c, b, load_c, gather_c, drain_b):
      wait_gathers(b)
      compute(b)
      issue_scatters(b)
      if drain_b is not None:
        wait_scatters(drain_b)
      if load_c is not None:
        load_idx(load_c[0], load_c[1])
      if gather_c is not None:
        wait_idx(gather_c[0], gather_c[1])
        issue_gathers(gather_c[1])

    step(0, 0, (3, 3), (2, 2), None)
    step(1, 1, (4, 0), (3, 3), 0)
    step(2, 2, (5, 1), (4, 0), 1)
    step(3, 3, (6, 2), (5, 1), 2)

    def macro(j, carry):
      c = 4 * j
      step(c + 0, 0, (c + 3, 3), (c + 2, 2), 3)
      step(c + 1, 1, (c + 4, 0), (c + 3, 3), 0)
      step(c + 2, 2, (c + 5, 1), (c + 4, 0), 1)
      step(c + 3, 3, (c + 6, 2), (c + 5, 1), 2)
      return carry

    lax.fori_loop(1, _NCH // 4 - 1, macro, 0)

    c = _NCH - 4
    step(c + 0, 0, (c + 3, 3), (c + 2, 2), 3)
    step(c + 1, 1, None, (c + 3, 3), 0)
    step(c + 2, 2, None, None, 1)
    step(c + 3, 3, None, None, 2)
    wait_scatters(3)
    plsc.subcore_barrier()

    sl = pl.ds(sid * _RPT, _RPT)
    pltpu.sync_copy(acc_sh.at[sl], acc_hbm.at[cid, sl])

  return edge_kernel


_ROWS = 1000
_EB = 4096


def _pre_body(h_ref, wcat_ref, b1_ref, pa_ref, pb_ref):
  p = jnp.dot(h_ref[...], wcat_ref[...], preferred_element_type=jnp.float32)
  pa_ref[...] = p[:, :_D]
  pb_ref[...] = p[:, _D:] + b1_ref[...]


def _hist_body(d_ref, hist_ref):
  i = pl.program_id(0)
  d = d_ref[...]
  lanes = lax.broadcasted_iota(jnp.int32, (_EB, _D), 1)
  ohhi = (lax.shift_right_logical(d, 7) == lanes).astype(jnp.float32)
  ohlo = (lax.bitwise_and(d, 127) == lanes).astype(jnp.float32)
  p = lax.dot_general(ohhi, ohlo, (((0,), (0,)), ((), ())),
                      preferred_element_type=jnp.float32)

  @pl.when(i == 0)
  def _():
    hist_ref[...] = p

  @pl.when(i > 0)
  def _():
    hist_ref[...] += p


def _post_body(acc_ref, cnt_ref, h_ref, w2_ref, b2_ref, u1_ref, ub1_ref,
               u2_ref, ub2_ref, out_ref):
  accs = acc_ref[0] + acc_ref[1]
  agg = (jnp.dot(accs, w2_ref[...], preferred_element_type=jnp.float32)
         + cnt_ref[...] * b2_ref[...])
  u1 = u1_ref[...]
  u = (jnp.dot(h_ref[...], u1[:_D], preferred_element_type=jnp.float32)
       + jnp.dot(agg, u1[_D:], preferred_element_type=jnp.float32)
       + ub1_ref[...])
  t = u / (1.0 + jnp.exp(-u))
  out_ref[...] = (jnp.dot(t, u2_ref[...], preferred_element_type=jnp.float32)
                  + ub2_ref[...])


def kernel(h, edge_index, msg_w1, msg_b1, msg_w2, msg_b2,
           upd_w1, upd_b1, upd_w2, upd_b2):
  src = edge_index[0].astype(jnp.int32)
  dst = edge_index[1].astype(jnp.int32)
  pad = _NEP - _NE
  iota_pad = jnp.arange(pad, dtype=jnp.int32)
  src_p = jnp.concatenate([src, iota_pad % _NN])
  dst_p = jnp.concatenate([dst, _NN + iota_pad % (_NNP - _NN)])

  wcat = -jnp.concatenate([msg_w1[:_D], msg_w1[_D:]], axis=1)
  msg_b1 = -msg_b1

  pa, pb = pl.pallas_call(
      _pre_body,
      grid=(_NN // _ROWS,),
      in_specs=[
          pl.BlockSpec((_ROWS, _D), lambda i: (i, 0)),
          pl.BlockSpec((_D, 2 * _D), lambda i: (0, 0)),
          pl.BlockSpec((1, _D), lambda i: (0, 0)),
      ],
      out_specs=[
          pl.BlockSpec((_ROWS, _D), lambda i: (i, 0)),
          pl.BlockSpec((_ROWS, _D), lambda i: (i, 0)),
      ],
      out_shape=[
          jax.ShapeDtypeStruct((_NN, _D), jnp.float32),
          jax.ShapeDtypeStruct((_NN, _D), jnp.float32),
      ],
  )(h, wcat, msg_b1[None, :])

  zeros = jnp.zeros((_RPT, _D), jnp.float32)
  acc = _make_edge_kernel()(pa, pb, src_p, dst_p, zeros)

  hist = pl.pallas_call(
      _hist_body,
      grid=(_NEP // _EB,),
      in_specs=[pl.BlockSpec((_EB, 1), lambda i: (i, 0))],
      out_specs=pl.BlockSpec((_D, _D), lambda i: (0, 0)),
      out_shape=jax.ShapeDtypeStruct((_D, _D), jnp.float32),
  )(dst_p.reshape(_NEP, 1))
  cnt_col = hist.reshape(_D * _D)[:_NN][:, None]

  out = pl.pallas_call(
      _post_body,
      grid=(_NN // _ROWS,),
      in_specs=[
          pl.BlockSpec((2, _ROWS, _D), lambda i: (0, i, 0)),
          pl.BlockSpec((_ROWS, 1), lambda i: (i, 0)),
          pl.BlockSpec((_ROWS, _D), lambda i: (i, 0)),
          pl.BlockSpec((_D, _D), lambda i: (0, 0)),
          pl.BlockSpec((1, _D), lambda i: (0, 0)),
          pl.BlockSpec((2 * _D, _D), lambda i: (0, 0)),
          pl.BlockSpec((1, _D), lambda i: (0, 0)),
          pl.BlockSpec((_D, _D), lambda i: (0, 0)),
          pl.BlockSpec((1, _D), lambda i: (0, 0)),
      ],
      out_specs=pl.BlockSpec((_ROWS, _D), lambda i: (i, 0)),
      out_shape=jax.ShapeDtypeStruct((_NN, _D), jnp.float32),
  )(acc, cnt_col, h, msg_w2, msg_b2[None, :], upd_w1, upd_b1[None, :],
    upd_w2, upd_b2[None, :])
  return out
